# baseline (device time: 23854 ns/iter reference)
import jax
import jax.numpy as jnp
from jax import lax
from jax.experimental import pallas as pl
from jax.experimental.pallas import tpu as pltpu

N_DEV = 16


def kernel(x, Wq, K_ext, V_ext, Wo):
    B, Sq, D = x.shape
    _, Skv, Hq, Dh = K_ext.shape
    Dm = Hq * Dh
    Do = Wo.shape[1]
    ROWS = Sq // N_DEV
    PR = ROWS + 1

    Kf = K_ext.reshape(B, Skv, Dm)
    Vf = V_ext.reshape(B, Skv, Dm)

    def body(x_ref, wq_ref, k_ref, v_ref, wo_ref, out_ref,
             accP, recvP, accR, ybuf, s1, r1, s2, r2):
        me = lax.axis_index("i")
        bf16 = jnp.bfloat16

        barrier = pltpu.get_barrier_semaphore()
        for j in range(N_DEV):
            @pl.when(j != me)
            def _(j=j):
                pl.semaphore_signal(
                    barrier, inc=1, device_id=(j,),
                    device_id_type=pl.DeviceIdType.MESH)
        pl.semaphore_wait(barrier, N_DEV - 1)

        wq16 = wq_ref[...].astype(bf16)
        qb = lax.broadcasted_iota(jnp.int32, (Sq, Skv), 0) // 64
        kb = lax.broadcasted_iota(jnp.int32, (Sq, Skv), 1) // 64
        maskf = jnp.where(qb == kb, 1.0, 0.0).astype(jnp.float32)
        Qms = [lax.dot_general(x_ref[b].astype(bf16), wq16,
                               (((1,), (0,)), ((), ())),
                               preferred_element_type=jnp.float32
                               ).astype(bf16) for b in range(B)]

        for b in range(B):
            ses = []
            for h in range(Hq):
                cols = pl.ds(h * Dh, Dh)
                Kh = k_ref[b, :, cols].astype(bf16)
                Vh = v_ref[b, :, cols].astype(bf16)
                S = lax.dot_general(
                    Qms[b][:, h * Dh:(h + 1) * Dh], Kh,
                    (((1,), (1,)), ((), ())),
                    preferred_element_type=jnp.float32)
                w = jnp.exp(S * 0.125) * maskf
                ses.append(jnp.sum(w, axis=1))
                ctx = lax.dot_general(
                    w.astype(bf16), Vh, (((1,), (0,)), ((), ())),
                    preferred_element_type=jnp.float32)
                accP[b, :, 0:ROWS, cols] = (
                    ctx.astype(bf16).reshape(N_DEV, ROWS, Dh))
            E1 = jnp.stack(ses, axis=1).reshape(N_DEV, ROWS, Hq)
            E2 = jnp.transpose(E1, (0, 2, 1)).reshape(N_DEV, Hq * ROWS)
            accP[b, :, ROWS, pl.ds(0, Hq * ROWS)] = E2.astype(bf16)

        for j in range(N_DEV):
            @pl.when(j != me)
            def _(j=j):
                pltpu.make_async_remote_copy(
                    src_ref=accP.at[:, j, :, :],
                    dst_ref=recvP.at[me],
                    send_sem=s1.at[j], recv_sem=r1.at[me],
                    device_id=(j,),
                    device_id_type=pl.DeviceIdType.MESH).start()

            @pl.when(j == me)
            def _(j=j):
                accR[...] = accP[:, j, :, :].astype(jnp.float32)

        for j in range(N_DEV):
            @pl.when(j != me)
            def _(j=j):
                pltpu.make_async_remote_copy(
                    src_ref=accP.at[:, j, :, :],
                    dst_ref=recvP.at[j],
                    send_sem=s1.at[j], recv_sem=r1.at[j],
                    device_id=(j,), device_id_type=pl.DeviceIdType.MESH,
                ).wait_recv()
                accR[...] += recvP[j].astype(jnp.float32)

        C = accR[:, 0:ROWS, :]
        Efl = accR[:, ROWS, pl.ds(0, Hq * ROWS)]
        Ehr = jnp.transpose(
            Efl.reshape(B, Hq, ROWS), (0, 2, 1))
        Nrm = C.reshape(B, ROWS, Hq, Dh) / Ehr[..., None]
        ybuf[:, pl.ds(me * ROWS, ROWS), :] = (
            Nrm.reshape(B, ROWS, Dm).astype(bf16))

        for j in range(N_DEV):
            @pl.when(j != me)
            def _(j=j):
                pltpu.make_async_remote_copy(
                    src_ref=ybuf.at[:, pl.ds(me * ROWS, ROWS), :],
                    dst_ref=ybuf.at[:, pl.ds(me * ROWS, ROWS), :],
                    send_sem=s2.at[j], recv_sem=r2.at[me],
                    device_id=(j,),
                    device_id_type=pl.DeviceIdType.MESH).start()

        wo16 = wo_ref[...].astype(bf16)
        for b in range(B):
            out_ref[b, pl.ds(me * ROWS, ROWS), :] = lax.dot_general(
                ybuf[b, pl.ds(me * ROWS, ROWS), :], wo16,
                (((1,), (0,)), ((), ())),
                preferred_element_type=jnp.float32)

        for j in range(N_DEV):
            jsl = pl.ds(j * ROWS, ROWS)

            @pl.when(j != me)
            def _(j=j, jsl=jsl):
                pltpu.make_async_remote_copy(
                    src_ref=ybuf.at[:, jsl, :],
                    dst_ref=ybuf.at[:, jsl, :],
                    send_sem=s2.at[j], recv_sem=r2.at[j],
                    device_id=(j,), device_id_type=pl.DeviceIdType.MESH,
                ).wait_recv()
                for b in range(B):
                    out_ref[b, jsl, :] = lax.dot_general(
                        ybuf[b, jsl, :], wo16, (((1,), (0,)), ((), ())),
                        preferred_element_type=jnp.float32)

        for j in range(N_DEV):
            @pl.when(j != me)
            def _(j=j):
                pltpu.make_async_remote_copy(
                    src_ref=accP.at[:, j, :, :],
                    dst_ref=recvP.at[j],
                    send_sem=s1.at[j], recv_sem=r1.at[j],
                    device_id=(j,), device_id_type=pl.DeviceIdType.MESH,
                ).wait_send()
                pltpu.make_async_remote_copy(
                    src_ref=ybuf.at[:, pl.ds(0, ROWS), :],
                    dst_ref=ybuf.at[:, pl.ds(0, ROWS), :],
                    send_sem=s2.at[j], recv_sem=r2.at[j],
                    device_id=(j,), device_id_type=pl.DeviceIdType.MESH,
                ).wait_send()

    out_shape = jax.ShapeDtypeStruct((B, Sq, Do), jnp.float32)
    return pl.pallas_call(
        body,
        out_shape=out_shape,
        in_specs=[pl.BlockSpec(memory_space=pltpu.VMEM)] * 5,
        out_specs=pl.BlockSpec(memory_space=pltpu.VMEM),
        scratch_shapes=[
            pltpu.VMEM((B, N_DEV, PR, Dm), jnp.bfloat16),
            pltpu.VMEM((N_DEV, B, PR, Dm), jnp.bfloat16),
            pltpu.VMEM((B, PR, Dm), jnp.float32),
            pltpu.VMEM((B, Sq, Dm), jnp.bfloat16),
            pltpu.SemaphoreType.DMA((N_DEV,)),
            pltpu.SemaphoreType.DMA((N_DEV,)),
            pltpu.SemaphoreType.DMA((N_DEV,)),
            pltpu.SemaphoreType.DMA((N_DEV,)),
        ],
        compiler_params=pltpu.CompilerParams(collective_id=0),
    )(x, Wq, Kf, Vf, Wo)


# device time: 20430 ns/iter; 1.1676x vs baseline; 1.1676x over previous
import jax
import jax.numpy as jnp
from jax import lax
from jax.experimental import pallas as pl
from jax.experimental.pallas import tpu as pltpu

N_DEV = 16


def kernel(x, Wq, K_ext, V_ext, Wo):
    B, Sq, D = x.shape
    _, Skv, Hq, Dh = K_ext.shape
    Dm = Hq * Dh
    Do = Wo.shape[1]
    QB = Sq // 64
    ROWS = Sq // N_DEV
    OPB = 64 // ROWS

    Kf = K_ext.reshape(B, Skv, Dm)
    Vf = V_ext.reshape(B, Skv, Dm)

    def body(x_ref, wq_ref, k_ref, v_ref, wo_ref, out_ref,
             accW, recvW, accR, ybuf,
             s1C, r1C, s2, r2):
        me = lax.axis_index("i")

        barrier = pltpu.get_barrier_semaphore()
        for j in range(N_DEV):
            @pl.when(j != me)
            def _(j=j):
                pl.semaphore_signal(
                    barrier, inc=1, device_id=(j,),
                    device_id_type=pl.DeviceIdType.MESH)

        bf16 = jnp.bfloat16
        wq16 = wq_ref[...].astype(bf16)
        qb = lax.broadcasted_iota(jnp.int32, (Sq, Skv), 0) // 64
        kb = lax.broadcasted_iota(jnp.int32, (Sq, Skv), 1) // 64
        maskf = jnp.where(qb == kb, 1.0, 0.0).astype(jnp.float32)
        Qms = [lax.dot_general(x_ref[b].astype(bf16), wq16,
                               (((1,), (0,)), ((), ())),
                               preferred_element_type=jnp.float32
                               ).astype(bf16) for b in range(B)]

        for b in range(B):
            ses = []
            for h in range(Hq):
                cols = pl.ds(h * Dh, Dh)
                Kh = k_ref[b, :, cols].astype(bf16)
                Vh = v_ref[b, :, cols].astype(bf16)
                S = lax.dot_general(
                    Qms[b][:, h * Dh:(h + 1) * Dh], Kh,
                    (((1,), (1,)), ((), ())),
                    preferred_element_type=jnp.float32)
                w = jnp.exp(S * 0.125) * maskf
                ses.append(jnp.sum(w, axis=1))
                ctx = lax.dot_general(
                    w.astype(bf16), Vh, (((1,), (0,)), ((), ())),
                    preferred_element_type=jnp.float32)
                accW[b, :, cols] = ctx.astype(bf16)
            Epad = jnp.concatenate(
                [jnp.stack(ses, axis=1),
                 jnp.zeros((Sq, 128 - Hq), jnp.float32)], axis=1)
            accW[b, :, pl.ds(Dm, 128)] = Epad.astype(bf16)

        pl.semaphore_wait(barrier, N_DEV - 1)
        for q in range(QB):
            for t in range(OPB):
                owner = q * OPB + t
                osl = pl.ds(owner * ROWS, ROWS)

                @pl.when(owner != me)
                def _(owner=owner, osl=osl):
                    pltpu.make_async_remote_copy(
                        src_ref=accW.at[:, osl, :],
                        dst_ref=recvW.at[me],
                        send_sem=s1C.at[owner], recv_sem=r1C.at[me],
                        device_id=(owner,),
                        device_id_type=pl.DeviceIdType.MESH).start()

                @pl.when(owner == me)
                def _(osl=osl):
                    accR[...] = accW[:, osl, :].astype(jnp.float32)

        for j in range(N_DEV):
            @pl.when(j != me)
            def _(j=j):
                pltpu.make_async_remote_copy(
                    src_ref=accW.at[:, pl.ds(0, ROWS), :],
                    dst_ref=recvW.at[j],
                    send_sem=s1C.at[j], recv_sem=r1C.at[j],
                    device_id=(j,), device_id_type=pl.DeviceIdType.MESH,
                ).wait_recv()
                accR[...] += recvW[j].astype(jnp.float32)

        RC = accR[:, :, 0:Dm]
        RE = accR[:, :, pl.ds(Dm, Hq)]
        Nrm = RC.reshape(B, ROWS, Hq, Dh) / RE[..., None]
        ybuf[:, pl.ds(me * ROWS, ROWS), :] = (
            Nrm.reshape(B, ROWS, Dm).astype(jnp.bfloat16))

        for j in range(N_DEV):
            @pl.when(j != me)
            def _(j=j):
                pltpu.make_async_remote_copy(
                    src_ref=ybuf.at[:, pl.ds(me * ROWS, ROWS), :],
                    dst_ref=ybuf.at[:, pl.ds(me * ROWS, ROWS), :],
                    send_sem=s2.at[j], recv_sem=r2.at[me],
                    device_id=(j,),
                    device_id_type=pl.DeviceIdType.MESH).start()

        for j in range(N_DEV):
            jsl = pl.ds(j * ROWS, ROWS)

            @pl.when(j != me)
            def _(j=j, jsl=jsl):
                pltpu.make_async_remote_copy(
                    src_ref=ybuf.at[:, jsl, :],
                    dst_ref=ybuf.at[:, jsl, :],
                    send_sem=s2.at[j], recv_sem=r2.at[j],
                    device_id=(j,), device_id_type=pl.DeviceIdType.MESH,
                ).wait_recv()

        wo16 = wo_ref[...].astype(bf16)
        for b in range(B):
            out_ref[b, :, :] = lax.dot_general(
                ybuf[b], wo16, (((1,), (0,)), ((), ())),
                preferred_element_type=jnp.float32)

        for j in range(N_DEV):
            @pl.when(j != me)
            def _(j=j):
                pltpu.make_async_remote_copy(
                    src_ref=accW.at[:, pl.ds(0, ROWS), :],
                    dst_ref=recvW.at[j],
                    send_sem=s1C.at[j], recv_sem=r1C.at[j],
                    device_id=(j,), device_id_type=pl.DeviceIdType.MESH,
                ).wait_send()
                pltpu.make_async_remote_copy(
                    src_ref=ybuf.at[:, pl.ds(0, ROWS), :],
                    dst_ref=ybuf.at[:, pl.ds(0, ROWS), :],
                    send_sem=s2.at[j], recv_sem=r2.at[j],
                    device_id=(j,), device_id_type=pl.DeviceIdType.MESH,
                ).wait_send()

    out_shape = jax.ShapeDtypeStruct((B, Sq, Do), jnp.float32)
    return pl.pallas_call(
        body,
        out_shape=out_shape,
        in_specs=[pl.BlockSpec(memory_space=pltpu.VMEM)] * 5,
        out_specs=pl.BlockSpec(memory_space=pltpu.VMEM),
        scratch_shapes=[
            pltpu.VMEM((B, Sq, Dm + 128), jnp.bfloat16),
            pltpu.VMEM((N_DEV, B, ROWS, Dm + 128), jnp.bfloat16),
            pltpu.VMEM((B, ROWS, Dm + 128), jnp.float32),
            pltpu.VMEM((B, Sq, Dm), jnp.bfloat16),
            pltpu.SemaphoreType.DMA((N_DEV,)),
            pltpu.SemaphoreType.DMA((N_DEV,)),
            pltpu.SemaphoreType.DMA((N_DEV,)),
            pltpu.SemaphoreType.DMA((N_DEV,)),
        ],
        compiler_params=pltpu.CompilerParams(collective_id=0),
    )(x, Wq, Kf, Vf, Wo)
